# baseline (speedup 1.0000x reference)
import functools
import math

import jax
import jax.numpy as jnp
from jax import lax
from jax.experimental import pallas as pl
from jax.experimental.pallas import tpu as pltpu
from jax.experimental.pallas import tpu_sc as plsc

N = 10000
E = 320000
F = 128
ALPHA = 0.5
LAMBDA = 1.0

NW = 32
NS = 16
CH = 128
K = 80
EW = K * CH
EP = NW * EW
NP = NS * 632
RPT = NP // NS

_mesh = plsc.VectorSubcoreMesh(core_axis_name="c", subcore_axis_name="s")
f32 = jnp.float32
i32 = jnp.int32


EPC = EP // NS
KC = EPC // CH


def _deg_body(idx2_hbm, zerosf_hbm, onesf_hbm, out_hbm, cidx, ones_v, acc):
    c = lax.axis_index("c")
    s = lax.axis_index("s")
    rbase = s * RPT
    pltpu.sync_copy(zerosf_hbm.at[pl.ds(rbase, RPT)], acc.at[pl.ds(rbase, RPT)])
    pltpu.sync_copy(onesf_hbm, ones_v)
    plsc.subcore_barrier()

    ebase = s * EPC

    def body(j, carry):
        b = pl.multiple_of(ebase + j * CH, 8)
        pltpu.sync_copy(idx2_hbm.at[c, pl.ds(b, CH)], cidx)
        pltpu.sync_copy(ones_v, acc.at[cidx], add=True)
        return carry

    lax.fori_loop(0, KC, body, 0)
    plsc.subcore_barrier()
    pltpu.sync_copy(acc.at[pl.ds(rbase, RPT)], out_hbm.at[c, pl.ds(rbase, RPT)])


_DEG_OUT = jax.ShapeDtypeStruct((2, NP, F), f32)
_DEG_SCRATCH = [
    pltpu.VMEM((CH,), i32),
    pltpu.VMEM((CH, F), f32),
    pltpu.VMEM_SHARED((NP, F), f32),
]
_deg_kernel = pl.kernel(_deg_body, out_type=_DEG_OUT, mesh=_mesh,
                        scratch_types=_DEG_SCRATCH)


def _agg_body(f_hbm, srcg_hbm, dstp_hbm, zerosf_hbm,
              out_hbm, sidx_all, didx0, didx1, buf0, buf1, acc, sem0, sem1):
    c = lax.axis_index("c")
    s = lax.axis_index("s")
    wid = s * 2 + c
    rbase = s * RPT
    pltpu.sync_copy(zerosf_hbm.at[pl.ds(rbase, RPT)], acc.at[pl.ds(rbase, RPT)])
    ebase = wid * EW
    pltpu.sync_copy(srcg_hbm.at[pl.ds(ebase, EW)], sidx_all)
    plsc.subcore_barrier()

    didx = (didx0, didx1)
    buf = (buf0, buf1)
    sem = (sem0, sem1)

    for b in range(2):
        pltpu.sync_copy(dstp_hbm.at[pl.ds(pl.multiple_of(ebase + b * CH, 8), CH)],
                        didx[b])
        pltpu.async_copy(f_hbm.at[sidx_all.at[pl.ds(b * CH, CH)]], buf[b], sem[b])

    def body(g, carry):
        for b in range(2):
            j = g * 2 + b
            pltpu.make_async_copy(zerosf_hbm.at[pl.ds(0, CH)], buf[b],
                                  sem[b]).wait()
            pltpu.sync_copy(buf[b], acc.at[didx[b]], add=True)
            jn = j + 2

            @pl.when(jn < K)
            def _():
                bn = pl.multiple_of(ebase + jn * CH, 8)
                pltpu.sync_copy(dstp_hbm.at[pl.ds(bn, CH)], didx[b])
                o = pl.multiple_of(jn * CH, 8)
                pltpu.async_copy(f_hbm.at[sidx_all.at[pl.ds(o, CH)]],
                                 buf[b], sem[b])
        return carry

    lax.fori_loop(0, K // 2, body, 0)
    plsc.subcore_barrier()
    pltpu.sync_copy(acc.at[pl.ds(rbase, RPT)], out_hbm.at[c, pl.ds(rbase, RPT)])


_AGG_OUT = jax.ShapeDtypeStruct((2, NP, F), f32)
_AGG_SCRATCH = [
    pltpu.VMEM((EW,), i32),
    pltpu.VMEM((CH,), i32),
    pltpu.VMEM((CH,), i32),
    pltpu.VMEM((CH, F), f32),
    pltpu.VMEM((CH, F), f32),
    pltpu.VMEM_SHARED((NP, F), f32),
    pltpu.SemaphoreType.DMA,
    pltpu.SemaphoreType.DMA,
]
_agg_kernel = pl.kernel(_agg_body, out_type=_AGG_OUT, mesh=_mesh,
                        scratch_types=_AGG_SCRATCH)


_R = 1000


def _dot(a, b):
    return jnp.dot(a, b, preferred_element_type=f32,
                   precision=jax.lax.Precision.HIGHEST)


def _prep_body(x_ref, wt_ref, b_ref, deg_ref,
               h_ref, f0_ref, ns_ref, nd_ref):
    ns = lax.rsqrt(deg_ref[0, :, 0:1] + 1.0)
    nd = lax.rsqrt(deg_ref[1, :, 0:1] + 1.0)
    h = jnp.maximum(_dot(x_ref[...], wt_ref[...]) + b_ref[...], 0.0)
    h_ref[...] = h
    f0_ref[...] = h * ns
    ns_ref[...] = ns
    nd_ref[...] = nd


def _rest0_body(p_ref, f_ref, h_ref, nd_ref, ns_ref, w1_ref, b_ref, fn_ref,
                *, beta):
    agg = p_ref[0] + p_ref[1] + f_ref[...]
    f = 0.5 * (agg * nd_ref[...] + h_ref[...])
    rst = (1.0 - beta) * f + beta * _dot(f, w1_ref[...]) + b_ref[...]
    fn_ref[...] = rst * ns_ref[...]


def _rest1_body(p_ref, f_ref, h_ref, nd_ref, w1_ref, b_ref,
                wdt_ref, bd_ref, out_ref, *, beta):
    agg = p_ref[0] + p_ref[1] + f_ref[...]
    f = 0.5 * (agg * nd_ref[...] + h_ref[...])
    rst = (1.0 - beta) * f + beta * _dot(f, w1_ref[...]) + b_ref[...]
    out_ref[...] = jnp.maximum(_dot(rst, wdt_ref[...]) + bd_ref[...], 0.0)


_spec_rows = pl.BlockSpec((_R, F), lambda i: (i, 0))
_spec_full = pl.BlockSpec((F, F), lambda i: (0, 0))
_spec_brow = pl.BlockSpec((1, F), lambda i: (0, 0))
_spec_deg = pl.BlockSpec((2, _R, F), lambda i: (0, i, 0))
_spec_p = pl.BlockSpec((2, _R, F), lambda i: (0, i, 0))
_spec_n1 = pl.BlockSpec((_R, 1), lambda i: (i, 0))

_prep_call = pl.pallas_call(
    _prep_body,
    grid=(N // _R,),
    in_specs=[_spec_rows, _spec_full, _spec_brow, _spec_deg],
    out_specs=[_spec_rows, _spec_rows, _spec_n1, _spec_n1],
    out_shape=[
        jax.ShapeDtypeStruct((N, F), f32),
        jax.ShapeDtypeStruct((N, F), f32),
        jax.ShapeDtypeStruct((N, 1), f32),
        jax.ShapeDtypeStruct((N, 1), f32),
    ],
)


def _make_rest0(beta):
    return pl.pallas_call(
        functools.partial(_rest0_body, beta=beta),
        grid=(N // _R,),
        in_specs=[_spec_p, _spec_rows, _spec_rows, _spec_n1, _spec_n1,
                  _spec_full, _spec_brow],
        out_specs=_spec_rows,
        out_shape=jax.ShapeDtypeStruct((N, F), f32),
    )


def _make_rest1(beta):
    return pl.pallas_call(
        functools.partial(_rest1_body, beta=beta),
        grid=(N // _R,),
        in_specs=[_spec_p, _spec_rows, _spec_rows, _spec_n1,
                  _spec_full, _spec_brow, _spec_full, _spec_brow],
        out_specs=_spec_rows,
        out_shape=jax.ShapeDtypeStruct((N, F), f32),
    )


_BETA0 = math.log(LAMBDA / 1.0 + 1.0)
_BETA1 = math.log(LAMBDA / 2.0 + 1.0)
_rest0_call = _make_rest0(_BETA0)
_rest1_call = _make_rest1(_BETA1)


def kernel(x, edge_index, W_enc, b_enc, w1_l0, b_l0, w1_l1, b_l1, W_dec, b_dec):
    src = edge_index[0]
    dst = edge_index[1]
    pad = EP - E
    srcg = jnp.concatenate([src, jnp.zeros((pad,), i32)])
    srcd = jnp.concatenate([src, jnp.full((pad,), N, i32)])
    dstp = jnp.concatenate([dst, jnp.full((pad,), N, i32)])
    zerosf = jnp.zeros((NP, F), f32)
    onesf = jnp.ones((CH, F), f32)
    idx2 = jnp.stack([srcd, dstp])

    deg = _deg_kernel(idx2, zerosf, onesf)
    h, f0, ns, nd = _prep_call(x, W_enc.T, b_enc.reshape(1, F), deg)
    p0 = _agg_kernel(f0, srcg, dstp, zerosf)
    f1 = _rest0_call(p0, f0, h, nd, ns, w1_l0, b_l0.reshape(1, F))
    p1 = _agg_kernel(f1, srcg, dstp, zerosf)
    out = _rest1_call(p1, f1, h, nd, w1_l1, b_l1.reshape(1, F),
                      W_dec.T, b_dec.reshape(1, F))
    return out


# speedup vs baseline: 2.1798x; 2.1798x over previous
import functools
import math

import jax
import jax.numpy as jnp
from jax import lax
from jax.experimental import pallas as pl
from jax.experimental.pallas import tpu as pltpu
from jax.experimental.pallas import tpu_sc as plsc

N = 10000
E = 320000
F = 128
ALPHA = 0.5
LAMBDA = 1.0

NW = 32
NS = 16
CH = 128
K = 80
EW = K * CH
EP = NW * EW
NP = NS * 632
RPT = NP // NS

_mesh = plsc.VectorSubcoreMesh(core_axis_name="c", subcore_axis_name="s")
f32 = jnp.float32
i32 = jnp.int32


EPC = EP // NS
KC = EPC // CH


def _deg_body(idx2_hbm, zerosf_hbm, onesf_hbm, out_hbm, cidx, ones_v, acc):
    c = lax.axis_index("c")
    s = lax.axis_index("s")
    rbase = s * RPT
    pltpu.sync_copy(zerosf_hbm.at[pl.ds(rbase, RPT)], acc.at[pl.ds(rbase, RPT)])
    pltpu.sync_copy(onesf_hbm, ones_v)
    plsc.subcore_barrier()

    ebase = s * EPC

    def body(j, carry):
        b = pl.multiple_of(ebase + j * CH, 8)
        pltpu.sync_copy(idx2_hbm.at[c, pl.ds(b, CH)], cidx)
        pltpu.sync_copy(ones_v, acc.at[cidx], add=True)
        return carry

    lax.fori_loop(0, KC, body, 0)
    plsc.subcore_barrier()
    pltpu.sync_copy(acc.at[pl.ds(rbase, RPT)], out_hbm.at[c, pl.ds(rbase, RPT)])


_DEG_OUT = jax.ShapeDtypeStruct((2, NP, F), f32)
_DEG_SCRATCH = [
    pltpu.VMEM((CH,), i32),
    pltpu.VMEM((CH, F), f32),
    pltpu.VMEM_SHARED((NP, F), f32),
]
_deg_kernel = pl.kernel(_deg_body, out_type=_DEG_OUT, mesh=_mesh,
                        scratch_types=_DEG_SCRATCH)


def _agg_body(f_hbm, srcg_hbm, dstp_hbm, zerosf_hbm,
              out_hbm, sidx_all, didx0, didx1, buf0, buf1, acc, sem0, sem1):
    c = lax.axis_index("c")
    s = lax.axis_index("s")
    wid = s * 2 + c
    rbase = s * RPT
    pltpu.sync_copy(zerosf_hbm.at[pl.ds(rbase, RPT)], acc.at[pl.ds(rbase, RPT)])
    ebase = wid * EW
    pltpu.sync_copy(srcg_hbm.at[pl.ds(ebase, EW)], sidx_all)
    plsc.subcore_barrier()

    didx = (didx0, didx1)
    buf = (buf0, buf1)
    sem = (sem0, sem1)

    for b in range(2):
        pltpu.sync_copy(dstp_hbm.at[pl.ds(pl.multiple_of(ebase + b * CH, 8), CH)],
                        didx[b])
        pltpu.async_copy(f_hbm.at[sidx_all.at[pl.ds(b * CH, CH)]], buf[b], sem[b])

    def body(g, carry):
        for b in range(2):
            j = g * 2 + b
            pltpu.make_async_copy(zerosf_hbm.at[pl.ds(0, CH)], buf[b],
                                  sem[b]).wait()
            pltpu.sync_copy(buf[b], acc.at[didx[b]], add=True)
            jn = j + 2

            @pl.when(jn < K)
            def _():
                bn = pl.multiple_of(ebase + jn * CH, 8)
                pltpu.sync_copy(dstp_hbm.at[pl.ds(bn, CH)], didx[b])
                o = pl.multiple_of(jn * CH, 8)
                pltpu.async_copy(f_hbm.at[sidx_all.at[pl.ds(o, CH)]],
                                 buf[b], sem[b])
        return carry

    lax.fori_loop(0, K // 2, body, 0)
    plsc.subcore_barrier()
    pltpu.sync_copy(acc.at[pl.ds(rbase, RPT)], out_hbm.at[c, pl.ds(rbase, RPT)])


_AGG_OUT = jax.ShapeDtypeStruct((2, NP, F), f32)
_AGG_SCRATCH = [
    pltpu.VMEM((EW,), i32),
    pltpu.VMEM((CH,), i32),
    pltpu.VMEM((CH,), i32),
    pltpu.VMEM((CH, F), f32),
    pltpu.VMEM((CH, F), f32),
    pltpu.VMEM_SHARED((NP, F), f32),
    pltpu.SemaphoreType.DMA,
    pltpu.SemaphoreType.DMA,
]
_agg_kernel = pl.kernel(_agg_body, out_type=_AGG_OUT, mesh=_mesh,
                        scratch_types=_AGG_SCRATCH)


_R = 1000


def _dot(a, b):
    return jnp.dot(a, b, preferred_element_type=f32,
                   precision=jax.lax.Precision.HIGHEST)


def _prep_body(x_ref, wt_ref, b_ref, deg_ref,
               h_ref, f0_ref, ns_ref, nd_ref):
    ns = lax.rsqrt(deg_ref[0, :, 0:1] + 1.0)
    nd = lax.rsqrt(deg_ref[1, :, 0:1] + 1.0)
    h = jnp.maximum(_dot(x_ref[...], wt_ref[...]) + b_ref[...], 0.0)
    h_ref[...] = h
    f0_ref[...] = h * ns
    ns_ref[...] = ns
    nd_ref[...] = nd


def _rest0_body(p_ref, f_ref, h_ref, nd_ref, ns_ref, w1_ref, b_ref, fn_ref,
                *, beta):
    agg = p_ref[0] + p_ref[1] + f_ref[...]
    f = 0.5 * (agg * nd_ref[...] + h_ref[...])
    rst = (1.0 - beta) * f + beta * _dot(f, w1_ref[...]) + b_ref[...]
    fn_ref[...] = rst * ns_ref[...]


def _rest1_body(p_ref, f_ref, h_ref, nd_ref, w1_ref, b_ref,
                wdt_ref, bd_ref, out_ref, *, beta):
    agg = p_ref[0] + p_ref[1] + f_ref[...]
    f = 0.5 * (agg * nd_ref[...] + h_ref[...])
    rst = (1.0 - beta) * f + beta * _dot(f, w1_ref[...]) + b_ref[...]
    out_ref[...] = jnp.maximum(_dot(rst, wdt_ref[...]) + bd_ref[...], 0.0)


_spec_rows = pl.BlockSpec((_R, F), lambda i: (i, 0))
_spec_full = pl.BlockSpec((F, F), lambda i: (0, 0))
_spec_brow = pl.BlockSpec((1, F), lambda i: (0, 0))
_spec_deg = pl.BlockSpec((2, _R, F), lambda i: (0, i, 0))
_spec_p = pl.BlockSpec((2, _R, F), lambda i: (0, i, 0))
_spec_n1 = pl.BlockSpec((_R, 1), lambda i: (i, 0))

_prep_call = pl.pallas_call(
    _prep_body,
    grid=(N // _R,),
    in_specs=[_spec_rows, _spec_full, _spec_brow, _spec_deg],
    out_specs=[_spec_rows, _spec_rows, _spec_n1, _spec_n1],
    out_shape=[
        jax.ShapeDtypeStruct((N, F), f32),
        jax.ShapeDtypeStruct((N, F), f32),
        jax.ShapeDtypeStruct((N, 1), f32),
        jax.ShapeDtypeStruct((N, 1), f32),
    ],
)


def _make_rest0(beta):
    return pl.pallas_call(
        functools.partial(_rest0_body, beta=beta),
        grid=(N // _R,),
        in_specs=[_spec_p, _spec_rows, _spec_rows, _spec_n1, _spec_n1,
                  _spec_full, _spec_brow],
        out_specs=_spec_rows,
        out_shape=jax.ShapeDtypeStruct((N, F), f32),
    )


def _make_rest1(beta):
    return pl.pallas_call(
        functools.partial(_rest1_body, beta=beta),
        grid=(N // _R,),
        in_specs=[_spec_p, _spec_rows, _spec_rows, _spec_n1,
                  _spec_full, _spec_brow, _spec_full, _spec_brow],
        out_specs=_spec_rows,
        out_shape=jax.ShapeDtypeStruct((N, F), f32),
    )


_BETA0 = math.log(LAMBDA / 1.0 + 1.0)
_BETA1 = math.log(LAMBDA / 2.0 + 1.0)
_rest0_call = _make_rest0(_BETA0)
_rest1_call = _make_rest1(_BETA1)


def kernel(x, edge_index, W_enc, b_enc, w1_l0, b_l0, w1_l1, b_l1, W_dec, b_dec):
    src = edge_index[0]
    dst = edge_index[1]
    pad = EP - E
    pad_iota = jnp.arange(pad, dtype=i32)
    srcg = jnp.concatenate([src, pad_iota % N])
    srcd = jnp.concatenate([src, N + pad_iota % (NP - N)])
    dstp = jnp.concatenate([dst, N + pad_iota % (NP - N)])
    zerosf = jnp.zeros((NP, F), f32)
    onesf = jnp.ones((CH, F), f32)
    idx2 = jnp.stack([srcd, dstp])

    deg = _deg_kernel(idx2, zerosf, onesf)
    h, f0, ns, nd = _prep_call(x, W_enc.T, b_enc.reshape(1, F), deg)
    p0 = _agg_kernel(f0, srcg, dstp, zerosf)
    f1 = _rest0_call(p0, f0, h, nd, ns, w1_l0, b_l0.reshape(1, F))
    p1 = _agg_kernel(f1, srcg, dstp, zerosf)
    out = _rest1_call(p1, f1, h, nd, w1_l1, b_l1.reshape(1, F),
                      W_dec.T, b_dec.reshape(1, F))
    return out
